# Initial kernel scaffold; baseline (speedup 1.0000x reference)
import jax
import jax.numpy as jnp
from jax.experimental import pallas as pl


def kernel(x):
    raise NotImplementedError("write your pallas kernel here")


# speedup vs baseline: 1.4773x; 1.4773x over previous
import jax
import jax.numpy as jnp
from jax.experimental import pallas as pl

_COL_A = 5
_COL_B = 1000
_ROWS = 8192
_COLS = 4096
_BLK = 512


def _swap_body(x_ref, o_ref):
    xv = x_ref[...]
    lane = jax.lax.broadcasted_iota(jnp.int32, xv.shape, 1)
    col_a = xv[:, _COL_A:_COL_A + 1]
    col_b = xv[:, _COL_B:_COL_B + 1]
    o_ref[...] = jnp.where(lane == _COL_A, col_b,
                           jnp.where(lane == _COL_B, col_a, xv))


def kernel(x):
    return pl.pallas_call(
        _swap_body,
        grid=(_ROWS // _BLK,),
        in_specs=[pl.BlockSpec((_BLK, _COLS), lambda i: (i, 0))],
        out_specs=pl.BlockSpec((_BLK, _COLS), lambda i: (i, 0)),
        out_shape=jax.ShapeDtypeStruct((_ROWS, _COLS), x.dtype),
    )(x)
